# baseline (speedup 1.0000x reference)
import functools

import jax
import jax.numpy as jnp
from jax import lax
from jax.experimental import pallas as pl
from jax.experimental.pallas import tpu as pltpu
from jax.experimental.pallas import tpu_sc as plsc

NC = 2
NS = 16
NW = NC * NS
K = 128


def _matmul_call(x_pad, W, bm):
    n_pad, d_in = x_pad.shape
    d_out = W.shape[1]

    def body(x_ref, w_ref, o_ref):
        o_ref[...] = jnp.dot(x_ref[...], w_ref[...],
                             preferred_element_type=jnp.float32)

    return pl.pallas_call(
        body,
        grid=(n_pad // bm,),
        in_specs=[
            pl.BlockSpec((bm, d_in), lambda i: (i, 0)),
            pl.BlockSpec((d_in, d_out), lambda i: (0, 0)),
        ],
        out_specs=pl.BlockSpec((bm, d_out), lambda i: (i, 0)),
        out_shape=jax.ShapeDtypeStruct((n_pad, d_out), jnp.float32),
    )(x_pad, W)


def _finalize_call(p, cnt, b2, bm):
    _, n_pad, d = p.shape

    def body(p_ref, c_ref, b_ref, o_ref):
        s = p_ref[0] + p_ref[1]
        deg = c_ref[0, :, 0:1] + c_ref[1, :, 0:1] - 1.0
        o_ref[...] = s * (1.0 / deg) + b_ref[...]

    return pl.pallas_call(
        body,
        grid=(n_pad // bm,),
        in_specs=[
            pl.BlockSpec((2, bm, d), lambda i: (0, i, 0)),
            pl.BlockSpec((2, bm, d), lambda i: (0, i, 0)),
            pl.BlockSpec((1, d), lambda i: (0, 0)),
        ],
        out_specs=pl.BlockSpec((bm, d), lambda i: (i, 0)),
        out_shape=jax.ShapeDtypeStruct((n_pad, d), jnp.float32),
    )(p, cnt, b2)


def _sc_aggregate(h_pad, src2, dst2, zrow, ones128, iota, n_pad, d, n_chunks):
    rpt = n_pad // NS
    n_full, rem = divmod(rpt, K)
    mesh = plsc.VectorSubcoreMesh(core_axis_name="c", subcore_axis_name="s")

    scratch = [
        pltpu.VMEM_SHARED((n_pad, d), jnp.float32),
        pltpu.VMEM((K, d), jnp.float32),
        pltpu.VMEM((K, d), jnp.float32),
        pltpu.VMEM((K,), jnp.int32),
        pltpu.VMEM((K,), jnp.int32),
        pltpu.VMEM((K,), jnp.int32),
        pltpu.VMEM((K,), jnp.int32),
        pltpu.VMEM((K,), jnp.int32),
        pltpu.VMEM((K,), jnp.int32),
        pltpu.VMEM((K,), jnp.int32),
        pltpu.VMEM((K,), jnp.int32),
        pltpu.SemaphoreType.DMA,
        pltpu.SemaphoreType.DMA,
        pltpu.SemaphoreType.DMA,
        pltpu.SemaphoreType.DMA,
        pltpu.SemaphoreType.DMA,
        pltpu.SemaphoreType.DMA,
        pltpu.SemaphoreType.DMA,
        pltpu.SemaphoreType.DMA,
        pltpu.SemaphoreType.DMA,
        pltpu.SemaphoreType.DMA,
        pltpu.SemaphoreType.DMA,
        pltpu.SemaphoreType.DMA,
    ]
    if rem:
        scratch.append(pltpu.VMEM((rem,), jnp.int32))

    @functools.partial(
        pl.kernel,
        out_type=[
            jax.ShapeDtypeStruct((NC * n_pad, d), jnp.float32),
            jax.ShapeDtypeStruct((NC * n_pad, d), jnp.float32),
        ],
        mesh=mesh,
        scratch_types=scratch,
    )
    def call(h_hbm, src_hbm, dst_hbm, zrow_hbm, ones_hbm, iota_hbm,
             p_hbm, cnt_hbm, acc, rows0, rows1,
             idxb0, idxb1, idxb2, idxb3, sidxb0, sidxb1, sidxb2, sidxb3,
             semId0, semId1, semId2, semId3, semIs0, semIs1, semIs2, semIs3,
             semG0, semG1, semS0, semS1, *idxr_opt):
        idxr = idxr_opt[0] if idxr_opt else None
        rows_b = [rows0, rows1]
        idx_b = [idxb0, idxb1, idxb2, idxb3]
        sidx_b = [sidxb0, sidxb1, sidxb2, sidxb3]
        semId = [semId0, semId1, semId2, semId3]
        semIs = [semIs0, semIs1, semIs2, semIs3]
        semG = [semG0, semG1]
        semS = [semS0, semS1]
        rows = rows0
        idxb = idxb0
        sidxb = sidxb0
        cid = lax.axis_index("c")
        sid = lax.axis_index("s")
        wid = cid * NS + sid
        rs = sid * rpt
        ebase = wid * n_chunks

        def scatter_slice_from_rows(seed_h):
            def body(c, carry):
                off = rs + c * K
                pltpu.sync_copy(iota_hbm.at[pl.ds(off, K)], idxb)
                if seed_h:
                    @pl.when(cid == 0)
                    def _():
                        pltpu.sync_copy(h_hbm.at[pl.ds(off, K)], rows)
                pltpu.sync_copy(rows, acc.at[idxb])
                return carry

            lax.fori_loop(0, n_full, body, 0)
            if rem:
                off = rs + n_full * K
                pltpu.sync_copy(iota_hbm.at[pl.ds(off, rem)], idxr)
                if seed_h:
                    @pl.when(cid == 0)
                    def _():
                        pltpu.sync_copy(h_hbm.at[pl.ds(off, rem)],
                                        rows.at[pl.ds(0, rem)])
                pltpu.sync_copy(rows.at[pl.ds(0, rem)], acc.at[idxr])

        def drain_slice_to(out_hbm):
            def body(c, carry):
                off = rs + c * K
                pltpu.sync_copy(iota_hbm.at[pl.ds(off, K)], idxb)
                pltpu.sync_copy(acc.at[idxb], rows)
                pltpu.sync_copy(rows, out_hbm.at[pl.ds(cid * n_pad + off, K)])
                return carry

            lax.fori_loop(0, n_full, body, 0)
            if rem:
                off = rs + n_full * K
                pltpu.sync_copy(iota_hbm.at[pl.ds(off, rem)], idxr)
                pltpu.sync_copy(acc.at[idxr], rows.at[pl.ds(0, rem)])
                pltpu.sync_copy(rows.at[pl.ds(0, rem)],
                                out_hbm.at[pl.ds(cid * n_pad + off, rem)])

        pltpu.sync_copy(ones_hbm, rows)
        scatter_slice_from_rows(seed_h=False)
        plsc.subcore_barrier()

        pltpu.async_copy(dst_hbm.at[ebase], idx_b[0], semId[0])
        pltpu.async_copy(dst_hbm.at[ebase + 1], idx_b[1], semId[1])

        def count_group(t, carry):
            for u in range(4):
                c = 4 * t + u
                p = u % 2
                q2 = (u + 2) % 4
                pltpu.make_async_copy(
                    dst_hbm.at[ebase + c], idx_b[u], semId[u]).wait()

                def _wait_s():
                    pltpu.make_async_copy(
                        rows0, acc.at[idx_b[q2]], semS[p]).wait()

                if u >= 2:
                    _wait_s()
                else:
                    pl.when(t > 0)(_wait_s)
                pltpu.async_copy(rows0, acc.at[idx_b[u]], semS[p], add=True)
                pltpu.async_copy(dst_hbm.at[ebase + c + 2], idx_b[q2],
                                 semId[q2])
            return carry

        lax.fori_loop(0, n_chunks // 4, count_group, 0)
        pltpu.make_async_copy(rows0, acc.at[idx_b[2]], semS[0]).wait()
        pltpu.make_async_copy(rows0, acc.at[idx_b[3]], semS[1]).wait()
        pltpu.make_async_copy(dst_hbm.at[ebase], idx_b[0], semId[0]).wait()
        pltpu.make_async_copy(dst_hbm.at[ebase], idx_b[1], semId[1]).wait()
        plsc.subcore_barrier()
        drain_slice_to(cnt_hbm)
        plsc.subcore_barrier()

        @pl.when(cid != 0)
        def _():
            pltpu.sync_copy(zrow_hbm, rows)

        scatter_slice_from_rows(seed_h=True)
        plsc.subcore_barrier()

        pltpu.async_copy(src_hbm.at[ebase], sidx_b[0], semIs[0])
        pltpu.async_copy(dst_hbm.at[ebase], idx_b[0], semId[0])
        pltpu.async_copy(src_hbm.at[ebase + 1], sidx_b[1], semIs[1])
        pltpu.async_copy(dst_hbm.at[ebase + 1], idx_b[1], semId[1])

        def edge_group(t, carry):
            for u in range(4):
                c = 4 * t + u
                p = u % 2
                q2 = (u + 2) % 4
                pltpu.make_async_copy(
                    src_hbm.at[ebase + c], sidx_b[u], semIs[u]).wait()
                pltpu.make_async_copy(
                    dst_hbm.at[ebase + c], idx_b[u], semId[u]).wait()

                def _wait_s():
                    pltpu.make_async_copy(
                        rows_b[p], acc.at[idx_b[q2]], semS[p]).wait()

                if u >= 2:
                    _wait_s()
                else:
                    pl.when(t > 0)(_wait_s)
                pltpu.async_copy(h_hbm.at[sidx_b[u]], rows_b[p], semG[p])
                pltpu.async_copy(src_hbm.at[ebase + c + 2], sidx_b[q2],
                                 semIs[q2])
                pltpu.async_copy(dst_hbm.at[ebase + c + 2], idx_b[q2],
                                 semId[q2])
                pltpu.make_async_copy(
                    h_hbm.at[sidx_b[u]], rows_b[p], semG[p]).wait()
                pltpu.async_copy(rows_b[p], acc.at[idx_b[u]], semS[p],
                                 add=True)
            return carry

        lax.fori_loop(0, n_chunks // 4, edge_group, 0)
        pltpu.make_async_copy(rows_b[0], acc.at[idx_b[2]], semS[0]).wait()
        pltpu.make_async_copy(rows_b[1], acc.at[idx_b[3]], semS[1]).wait()
        pltpu.make_async_copy(src_hbm.at[ebase], sidx_b[0], semIs[0]).wait()
        pltpu.make_async_copy(src_hbm.at[ebase], sidx_b[1], semIs[1]).wait()
        pltpu.make_async_copy(dst_hbm.at[ebase], idx_b[0], semId[0]).wait()
        pltpu.make_async_copy(dst_hbm.at[ebase], idx_b[1], semId[1]).wait()
        plsc.subcore_barrier()
        drain_slice_to(p_hbm)

    return call(h_pad, src2, dst2, zrow, ones128, iota)


def kernel(x, edge_index, W, b):
    n, d_in = x.shape
    d = W.shape[1]
    e = edge_index.shape[1]

    n_pad = -(-(n + 1) // (NS * 8)) * (NS * 8)
    e_pad = -(-e // (NW * K * 4)) * (NW * K * 4)
    pad = e_pad - e
    src = jnp.concatenate(
        [edge_index[0], jnp.zeros((pad + 2 * K,), jnp.int32)])
    dst = jnp.concatenate([
        edge_index[1],
        (n + (jnp.arange(pad, dtype=jnp.int32) % (n_pad - n))).astype(jnp.int32),
        jnp.zeros((2 * K,), jnp.int32),
    ])
    n_chunks = e_pad // (NW * K)
    src2 = src.reshape(NW * n_chunks + 2, K)
    dst2 = dst.reshape(NW * n_chunks + 2, K)

    x_pad = jnp.pad(x, ((0, n_pad - n), (0, 0)))
    h_pad = _matmul_call(x_pad, W, n_pad // NS)

    zrow = jnp.zeros((K, d), jnp.float32)
    ones128 = jnp.ones((K, d), jnp.float32)
    iota = jnp.arange(n_pad, dtype=jnp.int32)

    p, cnt = _sc_aggregate(h_pad, src2, dst2, zrow, ones128, iota,
                           n_pad, d, n_chunks)
    p = p.reshape(NC, n_pad, d)
    cnt = cnt.reshape(NC, n_pad, d)

    out = _finalize_call(p, cnt, b.reshape(1, d), n_pad // NS)
    return out[:n]


# speedup vs baseline: 1.1095x; 1.1095x over previous
import functools

import jax
import jax.numpy as jnp
from jax import lax
from jax.experimental import pallas as pl
from jax.experimental.pallas import tpu as pltpu
from jax.experimental.pallas import tpu_sc as plsc

NC = 2
NS = 16
NW = NC * NS
K = 128


def _matmul_call(x_pad, W, bm):
    n_pad, d_in = x_pad.shape
    d_out = W.shape[1]

    def body(x_ref, w_ref, o_ref):
        o_ref[...] = jnp.dot(x_ref[...], w_ref[...],
                             preferred_element_type=jnp.float32)

    return pl.pallas_call(
        body,
        grid=(n_pad // bm,),
        in_specs=[
            pl.BlockSpec((bm, d_in), lambda i: (i, 0)),
            pl.BlockSpec((d_in, d_out), lambda i: (0, 0)),
        ],
        out_specs=pl.BlockSpec((bm, d_out), lambda i: (i, 0)),
        out_shape=jax.ShapeDtypeStruct((n_pad, d_out), jnp.float32),
    )(x_pad, W)


def _finalize_call(p, cnt, b2, bm):
    _, n_pad, d = p.shape

    def body(p_ref, c_ref, b_ref, o_ref):
        s = p_ref[0] + p_ref[1]
        deg = c_ref[0, :, 0:1] + c_ref[1, :, 0:1] - 1.0
        o_ref[...] = s * (1.0 / deg) + b_ref[...]

    return pl.pallas_call(
        body,
        grid=(n_pad // bm,),
        in_specs=[
            pl.BlockSpec((2, bm, d), lambda i: (0, i, 0)),
            pl.BlockSpec((2, bm, d), lambda i: (0, i, 0)),
            pl.BlockSpec((1, d), lambda i: (0, 0)),
        ],
        out_specs=pl.BlockSpec((bm, d), lambda i: (i, 0)),
        out_shape=jax.ShapeDtypeStruct((n_pad, d), jnp.float32),
    )(p, cnt, b2)


def _sc_aggregate(h_pad, src2, dst2, zrow, ones128, iota, n_pad, d, n_chunks):
    rpt = n_pad // NS
    n_full, rem = divmod(rpt, K)
    mesh = plsc.VectorSubcoreMesh(core_axis_name="c", subcore_axis_name="s")

    scratch = [
        pltpu.VMEM_SHARED((n_pad, d), jnp.float32),
        pltpu.VMEM((K, d), jnp.float32),
        pltpu.VMEM((K,), jnp.int32),
        pltpu.VMEM((K,), jnp.int32),
    ]
    if rem:
        scratch.append(pltpu.VMEM((rem,), jnp.int32))

    @functools.partial(
        pl.kernel,
        out_type=[
            jax.ShapeDtypeStruct((NC * n_pad, d), jnp.float32),
            jax.ShapeDtypeStruct((NC * n_pad, d), jnp.float32),
        ],
        mesh=mesh,
        scratch_types=scratch,
    )
    def call(h_hbm, src_hbm, dst_hbm, zrow_hbm, ones_hbm, iota_hbm,
             p_hbm, cnt_hbm, acc, rows, idxb, sidxb, *idxr_opt):
        idxr = idxr_opt[0] if idxr_opt else None
        cid = lax.axis_index("c")
        sid = lax.axis_index("s")
        wid = cid * NS + sid
        rs = sid * rpt

        def scatter_slice_from_rows(seed_h):
            def body(c, carry):
                off = rs + c * K
                pltpu.sync_copy(iota_hbm.at[pl.ds(off, K)], idxb)
                if seed_h:
                    @pl.when(cid == 0)
                    def _():
                        pltpu.sync_copy(h_hbm.at[pl.ds(off, K)], rows)
                pltpu.sync_copy(rows, acc.at[idxb])
                return carry

            lax.fori_loop(0, n_full, body, 0)
            if rem:
                off = rs + n_full * K
                pltpu.sync_copy(iota_hbm.at[pl.ds(off, rem)], idxr)
                if seed_h:
                    @pl.when(cid == 0)
                    def _():
                        pltpu.sync_copy(h_hbm.at[pl.ds(off, rem)],
                                        rows.at[pl.ds(0, rem)])
                pltpu.sync_copy(rows.at[pl.ds(0, rem)], acc.at[idxr])

        def drain_slice_to(out_hbm):
            def body(c, carry):
                off = rs + c * K
                pltpu.sync_copy(iota_hbm.at[pl.ds(off, K)], idxb)
                pltpu.sync_copy(acc.at[idxb], rows)
                pltpu.sync_copy(rows, out_hbm.at[pl.ds(cid * n_pad + off, K)])
                return carry

            lax.fori_loop(0, n_full, body, 0)
            if rem:
                off = rs + n_full * K
                pltpu.sync_copy(iota_hbm.at[pl.ds(off, rem)], idxr)
                pltpu.sync_copy(acc.at[idxr], rows.at[pl.ds(0, rem)])
                pltpu.sync_copy(rows.at[pl.ds(0, rem)],
                                out_hbm.at[pl.ds(cid * n_pad + off, rem)])

        pltpu.sync_copy(ones_hbm, rows)
        scatter_slice_from_rows(seed_h=False)
        plsc.subcore_barrier()

        def count_chunk(c, carry):
            pltpu.sync_copy(dst_hbm.at[wid * n_chunks + c], idxb)
            pltpu.sync_copy(rows, acc.at[idxb], add=True)
            return carry

        lax.fori_loop(0, n_chunks, count_chunk, 0)
        plsc.subcore_barrier()
        drain_slice_to(cnt_hbm)
        plsc.subcore_barrier()

        @pl.when(cid != 0)
        def _():
            pltpu.sync_copy(zrow_hbm, rows)

        scatter_slice_from_rows(seed_h=True)
        plsc.subcore_barrier()

        def edge_chunk(c, carry):
            flat = wid * n_chunks + c
            pltpu.sync_copy(src_hbm.at[flat], sidxb)
            pltpu.sync_copy(dst_hbm.at[flat], idxb)
            pltpu.sync_copy(h_hbm.at[sidxb], rows)
            pltpu.sync_copy(rows, acc.at[idxb], add=True)
            return carry

        lax.fori_loop(0, n_chunks, edge_chunk, 0)
        plsc.subcore_barrier()
        drain_slice_to(p_hbm)

    return call(h_pad, src2, dst2, zrow, ones128, iota)


def kernel(x, edge_index, W, b):
    n, d_in = x.shape
    d = W.shape[1]
    e = edge_index.shape[1]

    n_pad = -(-(n + 1) // (NS * 8)) * (NS * 8)
    e1 = -(-e // NW) * NW
    g1 = e1 - e
    m = e1 // NW
    n_chunks = -(-m // K)
    p2 = n_chunks * K - m
    scrap0 = n + (jnp.arange(g1, dtype=jnp.int32) % (n_pad - n))
    src_t = jnp.concatenate(
        [edge_index[0], jnp.zeros((g1,), jnp.int32)]).reshape(NW, m)
    dst_t = jnp.concatenate([edge_index[1], scrap0]).reshape(NW, m)
    t_ids = jnp.arange(NW, dtype=jnp.int32)[:, None]
    i_ids = jnp.arange(p2, dtype=jnp.int32)[None, :]
    scrap = (n + (t_ids * 7 + i_ids) % (n_pad - n)).astype(jnp.int32)
    src_t = jnp.concatenate([src_t, jnp.zeros((NW, p2), jnp.int32)], axis=1)
    dst_t = jnp.concatenate([dst_t, scrap], axis=1)
    src2 = src_t.reshape(NW * n_chunks, K)
    dst2 = dst_t.reshape(NW * n_chunks, K)

    x_pad = jnp.pad(x, ((0, n_pad - n), (0, 0)))
    h_pad = _matmul_call(x_pad, W, n_pad // NS)

    zrow = jnp.zeros((K, d), jnp.float32)
    ones128 = jnp.ones((K, d), jnp.float32)
    iota = jnp.arange(n_pad, dtype=jnp.int32)

    p, cnt = _sc_aggregate(h_pad, src2, dst2, zrow, ones128, iota,
                           n_pad, d, n_chunks)
    p = p.reshape(NC, n_pad, d)
    cnt = cnt.reshape(NC, n_pad, d)

    out = _finalize_call(p, cnt, b.reshape(1, d), n_pad // NS)
    return out[:n]
